# baseline (speedup 1.0000x reference)
import functools

import jax
import jax.numpy as jnp
from jax import lax
from jax.experimental import pallas as pl
from jax.experimental.pallas import tpu as pltpu
from jax.experimental.pallas import tpu_sc as plsc

HIDDEN = 32
B = 16384
NC = 2
NS = 16
L = 16
NW = NC * NS
BPW = B // NW
G = BPW // L
CHUNK = 128
NCH = BPW // CHUNK


def _tec_kernel(p_h, p_t, p_r, n_h, n_t, n_r, ent, rel,
                p_out, n_out,
                iph, ipt, ipr, inh, int_, inr,
                rph, rpt, rpr, rnh, rnt, rnr,
                score_p, score_n, sem_p, sem_n):
    wid = lax.axis_index("s") * NC + lax.axis_index("c")
    base = wid * BPW

    pltpu.sync_copy(p_h.at[pl.ds(base, BPW)], iph)
    pltpu.sync_copy(p_t.at[pl.ds(base, BPW)], ipt)
    pltpu.sync_copy(p_r.at[pl.ds(base, BPW)], ipr)
    pltpu.sync_copy(n_h.at[pl.ds(base, BPW)], inh)
    pltpu.sync_copy(n_t.at[pl.ds(base, BPW)], int_)
    pltpu.sync_copy(n_r.at[pl.ds(base, BPW)], inr)

    copies_p = []
    copies_n = []
    for c in range(NCH):
        sl = pl.ds(c * CHUNK, CHUNK)
        copies_p.append(pltpu.async_copy(ent.at[iph.at[sl]], rph.at[sl], sem_p))
        copies_p.append(pltpu.async_copy(ent.at[ipt.at[sl]], rpt.at[sl], sem_p))
        copies_p.append(pltpu.async_copy(rel.at[ipr.at[sl]], rpr.at[sl], sem_p))
    for c in range(NCH):
        sl = pl.ds(c * CHUNK, CHUNK)
        copies_n.append(pltpu.async_copy(ent.at[inh.at[sl]], rnh.at[sl], sem_n))
        copies_n.append(pltpu.async_copy(ent.at[int_.at[sl]], rnt.at[sl], sem_n))
        copies_n.append(pltpu.async_copy(rel.at[inr.at[sl]], rnr.at[sl], sem_n))

    lane = lax.iota(jnp.int32, L)

    def _reduce(rh, rt, rr, score):
        def gbody(g, carry):
            row = g * L + lane
            acc = jnp.zeros((L,), jnp.float32)
            for j in range(HIDDEN):
                col = jnp.bitwise_and(lane + j, HIDDEN - 1)
                hv = plsc.load_gather(rh, [row, col])
                tv = plsc.load_gather(rt, [row, col])
                rv = plsc.load_gather(rr, [row, col])
                acc = acc + jnp.abs(hv + rv - tv)
            score[pl.ds(g * L, L)] = acc
            return carry
        lax.fori_loop(0, G, gbody, 0)

    for cp in copies_p:
        cp.wait()
    _reduce(rph, rpt, rpr, score_p)
    pltpu.sync_copy(score_p, p_out.at[pl.ds(base, BPW)])

    for cp in copies_n:
        cp.wait()
    _reduce(rnh, rnt, rnr, score_n)
    pltpu.sync_copy(score_n, n_out.at[pl.ds(base, BPW)])


@jax.jit
def kernel(p_h, p_t, p_r, n_h, n_t, n_r, ent_emb, rel_emb):
    mesh = plsc.VectorSubcoreMesh(core_axis_name="c", subcore_axis_name="s")
    f32 = jnp.float32
    i32 = jnp.int32
    run = pl.kernel(
        _tec_kernel,
        out_type=(jax.ShapeDtypeStruct((B,), f32),
                  jax.ShapeDtypeStruct((B,), f32)),
        mesh=mesh,
        scratch_types=(
            [pltpu.VMEM((BPW,), i32) for _ in range(6)]
            + [pltpu.VMEM((BPW, HIDDEN), f32) for _ in range(6)]
            + [pltpu.VMEM((BPW,), f32) for _ in range(2)]
            + [pltpu.SemaphoreType.DMA, pltpu.SemaphoreType.DMA]
        ),
        compiler_params=pltpu.CompilerParams(
            needs_layout_passes=False, use_tc_tiling_on_sc=False),
    )
    return run(p_h, p_t, p_r, n_h, n_t, n_r, ent_emb, rel_emb)
